# Initial kernel scaffold; baseline (speedup 1.0000x reference)
import jax
import jax.numpy as jnp
from jax.experimental import pallas as pl


def kernel(x, edge_index, etypes, norm, weight, w_comp, h_bias):
    raise NotImplementedError("write your pallas kernel here")


# speedup vs baseline: 2.7966x; 2.7966x over previous
import functools

import jax
import jax.numpy as jnp
from jax import lax
from jax.experimental import pallas as pl
from jax.experimental.pallas import tpu as pltpu
from jax.experimental.pallas import tpu_sc as plsc

NC = 2
NS = 16
LANES = 16
GRP = 128
STRIPE_G = 16


def _hall_body(wc_ref, w_ref, x_ref, out_ref):
    r = pl.program_id(0)
    nb = w_ref.shape[0]
    w_r = wc_ref[r, 0] * w_ref[0]
    for b in range(1, nb):
        w_r = w_r + wc_ref[r, b] * w_ref[b]
    out_ref[0] = jnp.dot(x_ref[...], w_r, preferred_element_type=jnp.float32)


def _compute_h_all(x, weight, w_comp):
    n, in_feat = x.shape
    nbases, _, out_feat = weight.shape
    nrels = w_comp.shape[0]
    bn = 2000
    grid = (nrels, n // bn)
    return pl.pallas_call(
        _hall_body,
        grid=grid,
        in_specs=[
            pl.BlockSpec(memory_space=pltpu.SMEM),
            pl.BlockSpec((nbases, in_feat, out_feat), lambda r, i: (0, 0, 0)),
            pl.BlockSpec((bn, in_feat), lambda r, i: (i, 0)),
        ],
        out_specs=pl.BlockSpec((1, bn, out_feat), lambda r, i: (r, i, 0)),
        out_shape=jax.ShapeDtypeStruct((nrels, n, out_feat), jnp.float32),
    )(w_comp, weight, x)


def _sc_body(n_nodes, half, gt, h2, srcp, etp, dstp, normp, biast, out,
             acc, srcb, etb, dstb, normb, fib, dstg, rows, sem):
    cc = lax.axis_index("c")
    ss = lax.axis_index("s")

    rows_per_tile = n_nodes // NS
    base = ss * rows_per_tile
    pltpu.sync_copy(biast.at[cc], rows)
    full, rem = divmod(rows_per_tile, GRP)
    for k in range(full):
        pltpu.sync_copy(rows, acc.at[pl.ds(base + k * GRP, GRP)])
    if rem:
        pltpu.sync_copy(rows.at[pl.ds(0, rem)],
                        acc.at[pl.ds(base + full * GRP, rem)])
    plsc.subcore_barrier()

    tile_e0 = ss * (gt * GRP)

    def stripe_body(st, _):
        sbase = tile_e0 + st * (STRIPE_G * GRP)
        ns = STRIPE_G * GRP
        pltpu.sync_copy(srcp.at[pl.ds(sbase, ns)], srcb)
        pltpu.sync_copy(etp.at[pl.ds(sbase, ns)], etb)
        pltpu.sync_copy(dstp.at[pl.ds(sbase, ns)], dstb)
        pltpu.sync_copy(normp.at[pl.ds(sbase, ns)], normb)

        def group_body(g, _):
            g0 = g * GRP
            for j in range(GRP // LANES):
                off = g0 + j * LANES
                et_v = etb[pl.ds(off, LANES)]
                s_v = srcb[pl.ds(off, LANES)]
                fib[pl.ds(j * LANES, LANES)] = (et_v * n_nodes + s_v) * 2 + cc
                dstg[pl.ds(j * LANES, LANES)] = dstb[pl.ds(off, LANES)]
            pltpu.async_copy(h2.at[fib], rows, sem).wait()

            def scale_body(b, _):
                nv = normb[pl.ds(g0 + b * LANES, LANES)]
                i0 = b * LANES
                for l in range(LANES):
                    nbc = lax.gather(
                        nv, jnp.full((LANES, 1), l, jnp.int32),
                        lax.GatherDimensionNumbers(
                            offset_dims=(), collapsed_slice_dims=(0,),
                            start_index_map=(0,)),
                        (1,), mode=lax.GatherScatterMode.PROMISE_IN_BOUNDS)
                    for j in range(half // LANES):
                        v = rows[i0 + l, pl.ds(j * LANES, LANES)]
                        rows[i0 + l, pl.ds(j * LANES, LANES)] = v * nbc
                return 0

            lax.fori_loop(0, GRP // LANES, scale_body, 0)
            pltpu.sync_copy(rows, acc.at[dstg], add=True)
            return 0

        lax.fori_loop(0, STRIPE_G, group_body, 0)
        return 0

    lax.fori_loop(0, gt // STRIPE_G, stripe_body, 0)
    plsc.subcore_barrier()

    pltpu.sync_copy(acc.at[pl.ds(base, rows_per_tile)],
                    out.at[pl.ds(base, rows_per_tile), cc])


def kernel(x, edge_index, etypes, norm, weight, w_comp, h_bias):
    n, in_feat = x.shape
    out_feat = weight.shape[2]
    nrels = w_comp.shape[0]
    e = etypes.shape[0]
    half = out_feat // 2

    h_all = _compute_h_all(x, weight, w_comp)
    h2 = h_all.reshape(nrels * n * 2, half)

    chunk = NS * STRIPE_G * GRP
    e_pad = ((e + chunk - 1) // chunk) * chunk
    pad = e_pad - e
    pad_iota = jnp.arange(pad, dtype=jnp.int32) % n
    srcp = jnp.concatenate([edge_index[0].astype(jnp.int32), pad_iota])
    dstp = jnp.concatenate([edge_index[1].astype(jnp.int32), pad_iota])
    etp = jnp.concatenate([etypes.astype(jnp.int32),
                           jnp.zeros((pad,), jnp.int32)])
    normp = jnp.concatenate([norm.reshape(e).astype(jnp.float32),
                             jnp.zeros((pad,), jnp.float32)])
    biast = jnp.broadcast_to(h_bias.reshape(2, 1, half), (2, GRP, half))
    gt = e_pad // (NS * GRP)

    mesh = plsc.VectorSubcoreMesh(core_axis_name="c", subcore_axis_name="s")
    sc_call = functools.partial(
        pl.kernel,
        out_type=jax.ShapeDtypeStruct((n, 2, half), jnp.float32),
        mesh=mesh,
        scratch_types=[
            pltpu.VMEM_SHARED((n, half), jnp.float32),
            pltpu.VMEM((STRIPE_G * GRP,), jnp.int32),
            pltpu.VMEM((STRIPE_G * GRP,), jnp.int32),
            pltpu.VMEM((STRIPE_G * GRP,), jnp.int32),
            pltpu.VMEM((STRIPE_G * GRP,), jnp.float32),
            pltpu.VMEM((GRP,), jnp.int32),
            pltpu.VMEM((GRP,), jnp.int32),
            pltpu.VMEM((GRP, half), jnp.float32),
            pltpu.SemaphoreType.DMA,
        ],
        compiler_params=pltpu.CompilerParams(use_tc_tiling_on_sc=False),
    )(functools.partial(_sc_body, n, half, gt))
    out3 = sc_call(h2, srcp, etp, dstp, normp, biast)
    return out3.reshape(n, out_feat)


# speedup vs baseline: 3.9430x; 1.4100x over previous
import functools

import jax
import jax.numpy as jnp
from jax import lax
from jax.experimental import pallas as pl
from jax.experimental.pallas import tpu as pltpu
from jax.experimental.pallas import tpu_sc as plsc

NC = 2
NS = 16
LANES = 16
GRP = 128
STRIPE_G = 32


def _hall_body(wc_ref, w_ref, x_ref, out_ref):
    r = pl.program_id(0)
    nb = w_ref.shape[0]
    w_r = wc_ref[r, 0] * w_ref[0]
    for b in range(1, nb):
        w_r = w_r + wc_ref[r, b] * w_ref[b]
    out_ref[0] = jnp.dot(x_ref[...], w_r, preferred_element_type=jnp.float32)


def _compute_h_all(x, weight, w_comp):
    n, in_feat = x.shape
    nbases, _, out_feat = weight.shape
    nrels = w_comp.shape[0]
    bn = 2000
    grid = (nrels, n // bn)
    return pl.pallas_call(
        _hall_body,
        grid=grid,
        in_specs=[
            pl.BlockSpec(memory_space=pltpu.SMEM),
            pl.BlockSpec((nbases, in_feat, out_feat), lambda r, i: (0, 0, 0)),
            pl.BlockSpec((bn, in_feat), lambda r, i: (i, 0)),
        ],
        out_specs=pl.BlockSpec((1, bn, out_feat), lambda r, i: (r, i, 0)),
        out_shape=jax.ShapeDtypeStruct((nrels, n, out_feat), jnp.float32),
    )(w_comp, weight, x)


def _sc_body(n_nodes, half, gt, h2, srcp, etp, dstp, normp, biast, out,
             acc, srcb, etb, dstb, normb,
             fib0, fib1, fib2, dstg0, dstg1, dstg2,
             normg0, normg1, normg2, rows0, rows1, rows2,
             semg0, semg1, semg2, sems0, sems1, sems2):
    cc = lax.axis_index("c")
    ss = lax.axis_index("s")
    fibs = (fib0, fib1, fib2)
    dstgs = (dstg0, dstg1, dstg2)
    normgs = (normg0, normg1, normg2)
    rowss = (rows0, rows1, rows2)
    semgs = (semg0, semg1, semg2)
    semss = (sems0, sems1, sems2)

    rows_per_tile = n_nodes // NS
    base = ss * rows_per_tile
    pltpu.sync_copy(biast.at[cc], rows0)
    full, rem = divmod(rows_per_tile, GRP)
    for k in range(full):
        pltpu.sync_copy(rows0, acc.at[pl.ds(base + k * GRP, GRP)])
    if rem:
        pltpu.sync_copy(rows0.at[pl.ds(0, rem)],
                        acc.at[pl.ds(base + full * GRP, rem)])
    plsc.subcore_barrier()

    te = gt * GRP
    tile_e0 = ss * te
    se = STRIPE_G * GRP

    def stage(s):
        sb = tile_e0 + s * se
        pltpu.sync_copy(srcp.at[pl.ds(sb, se)], srcb)
        pltpu.sync_copy(etp.at[pl.ds(sb, se)], etb)
        pltpu.sync_copy(dstp.at[pl.ds(sb, se)], dstb)
        pltpu.sync_copy(normp.at[pl.ds(sb, se)], normb)

    def prep(g, b):
        g0 = (g % STRIPE_G) * GRP
        for j in range(GRP // LANES):
            off = g0 + j * LANES
            et_v = etb[pl.ds(off, LANES)]
            s_v = srcb[pl.ds(off, LANES)]
            fibs[b][pl.ds(j * LANES, LANES)] = \
                (et_v * n_nodes + s_v) * 2 + cc
            dstgs[b][pl.ds(j * LANES, LANES)] = dstb[pl.ds(off, LANES)]
            normgs[b][pl.ds(j * LANES, LANES)] = normb[pl.ds(off, LANES)]

    def start_gather(b):
        pltpu.async_copy(h2.at[fibs[b]], rowss[b], semgs[b])

    def wait_gather(b):
        pltpu.make_async_copy(h2.at[fibs[b]], rowss[b], semgs[b]).wait()

    def scale(b):
        rows = rowss[b]

        def scale_body(q, _):
            nv = normgs[b][pl.ds(q * LANES, LANES)]
            i0 = q * LANES
            for l in range(LANES):
                nbc = lax.gather(
                    nv, jnp.full((LANES, 1), l, jnp.int32),
                    lax.GatherDimensionNumbers(
                        offset_dims=(), collapsed_slice_dims=(0,),
                        start_index_map=(0,)),
                    (1,), mode=lax.GatherScatterMode.PROMISE_IN_BOUNDS)
                for j in range(half // LANES):
                    v = rows[i0 + l, pl.ds(j * LANES, LANES)]
                    rows[i0 + l, pl.ds(j * LANES, LANES)] = v * nbc
            return 0

        lax.fori_loop(0, GRP // LANES, scale_body, 0)

    def start_scatter(b):
        pltpu.async_copy(rowss[b], acc.at[dstgs[b]], semss[b], add=True)

    def wait_scatter(b):
        pltpu.make_async_copy(rowss[b], acc.at[dstgs[b]], semss[b]).wait()

    stage(0)
    prep(0, 0)
    start_gather(0)
    steady = gt - 1
    assert steady % 3 == 0

    def triple(i3, _):
        for k in range(3):
            i = i3 + k
            b = k % 3

            @pl.when(i >= 2)
            def _():
                wait_scatter((b + 1) % 3)

            @pl.when((i + 1) % STRIPE_G == 0)
            def _():
                stage((i + 1) // STRIPE_G)
            prep(i + 1, (b + 1) % 3)
            start_gather((b + 1) % 3)
            wait_gather(b)
            scale(b)
            start_scatter(b)
        return 0

    lax.fori_loop(0, steady // 3, lambda t, c: triple(t * 3, c), 0)
    bl = (gt - 1) % 3
    wait_scatter((bl + 1) % 3)
    wait_gather(bl)
    scale(bl)
    start_scatter(bl)
    wait_scatter((bl + 2) % 3)
    wait_scatter(bl)
    plsc.subcore_barrier()

    pltpu.sync_copy(acc.at[pl.ds(base, rows_per_tile)],
                    out.at[pl.ds(base, rows_per_tile), cc])


def kernel(x, edge_index, etypes, norm, weight, w_comp, h_bias):
    n, in_feat = x.shape
    out_feat = weight.shape[2]
    nrels = w_comp.shape[0]
    e = etypes.shape[0]
    half = out_feat // 2

    h_all = _compute_h_all(x, weight, w_comp)
    h2 = h_all.reshape(nrels * n * 2, half)

    chunk = NS * STRIPE_G * GRP
    e_pad = ((e + chunk - 1) // chunk) * chunk
    pad = e_pad - e
    pad_iota = jnp.arange(pad, dtype=jnp.int32) % n
    srcp = jnp.concatenate([edge_index[0].astype(jnp.int32), pad_iota])
    dstp = jnp.concatenate([edge_index[1].astype(jnp.int32), pad_iota])
    etp = jnp.concatenate([etypes.astype(jnp.int32),
                           jnp.zeros((pad,), jnp.int32)])
    normp = jnp.concatenate([norm.reshape(e).astype(jnp.float32),
                             jnp.zeros((pad,), jnp.float32)])
    biast = jnp.broadcast_to(h_bias.reshape(2, 1, half), (2, GRP, half))
    gt = e_pad // (NS * GRP)

    mesh = plsc.VectorSubcoreMesh(core_axis_name="c", subcore_axis_name="s")
    sc_call = functools.partial(
        pl.kernel,
        out_type=jax.ShapeDtypeStruct((n, 2, half), jnp.float32),
        mesh=mesh,
        scratch_types=(
            [pltpu.VMEM_SHARED((n, half), jnp.float32)]
            + [pltpu.VMEM((STRIPE_G * GRP,), jnp.int32)] * 3
            + [pltpu.VMEM((STRIPE_G * GRP,), jnp.float32)]
            + [pltpu.VMEM((GRP,), jnp.int32)] * 6
            + [pltpu.VMEM((GRP,), jnp.float32)] * 3
            + [pltpu.VMEM((GRP, half), jnp.float32)] * 3
            + [pltpu.SemaphoreType.DMA] * 6
        ),
        compiler_params=pltpu.CompilerParams(use_tc_tiling_on_sc=False),
    )(functools.partial(_sc_body, n, half, gt))
    out3 = sc_call(h2, srcp, etp, dstp, normp, biast)
    return out3.reshape(n, out_feat)


# speedup vs baseline: 7.2551x; 1.8400x over previous
import functools

import jax
import jax.numpy as jnp
from jax import lax
from jax.experimental import pallas as pl
from jax.experimental.pallas import tpu as pltpu
from jax.experimental.pallas import tpu_sc as plsc

NC = 2
NS = 16
LANES = 16
GRP = 128
STRIPE_G = 32


def _hall_body(wc_ref, w_ref, x_ref, out_ref):
    r = pl.program_id(0)
    nb = w_ref.shape[0]
    w_r = wc_ref[r, 0] * w_ref[0]
    for b in range(1, nb):
        w_r = w_r + wc_ref[r, b] * w_ref[b]
    out_ref[0] = jnp.dot(x_ref[...], w_r, preferred_element_type=jnp.float32)


def _compute_h_all(x, weight, w_comp):
    n, in_feat = x.shape
    nbases, _, out_feat = weight.shape
    nrels = w_comp.shape[0]
    bn = 2000
    grid = (nrels, n // bn)
    return pl.pallas_call(
        _hall_body,
        grid=grid,
        in_specs=[
            pl.BlockSpec(memory_space=pltpu.SMEM),
            pl.BlockSpec((nbases, in_feat, out_feat), lambda r, i: (0, 0, 0)),
            pl.BlockSpec((bn, in_feat), lambda r, i: (i, 0)),
        ],
        out_specs=pl.BlockSpec((1, bn, out_feat), lambda r, i: (r, i, 0)),
        out_shape=jax.ShapeDtypeStruct((nrels, n, out_feat), jnp.float32),
    )(w_comp, weight, x)


def _sc_body(n_nodes, half, gt, h2, srcp, etp, dstp, normp, biast, out,
             acc, srcb, etb, dstb, normb,
             fib0, fib1, fib2, dstg0, dstg1, dstg2,
             normg0, normg1, normg2, rows0, rows1, rows2,
             semg0, semg1, semg2, sems0, sems1, sems2):
    cc = lax.axis_index("c")
    ss = lax.axis_index("s")
    fibs = (fib0, fib1, fib2)
    dstgs = (dstg0, dstg1, dstg2)
    normgs = (normg0, normg1, normg2)
    rowss = (rows0, rows1, rows2)
    semgs = (semg0, semg1, semg2)
    semss = (sems0, sems1, sems2)

    rows_per_tile = n_nodes // NS
    base = ss * rows_per_tile
    pltpu.sync_copy(biast.at[cc], rows0)
    full, rem = divmod(rows_per_tile, GRP)
    for k in range(full):
        pltpu.sync_copy(rows0, acc.at[pl.ds(base + k * GRP, GRP)])
    if rem:
        pltpu.sync_copy(rows0.at[pl.ds(0, rem)],
                        acc.at[pl.ds(base + full * GRP, rem)])
    plsc.subcore_barrier()

    te = gt * GRP
    tile_e0 = ss * te
    se = STRIPE_G * GRP

    def stage(s):
        sb = tile_e0 + s * se
        pltpu.sync_copy(srcp.at[pl.ds(sb, se)], srcb)
        pltpu.sync_copy(etp.at[pl.ds(sb, se)], etb)
        pltpu.sync_copy(dstp.at[pl.ds(sb, se)], dstb)
        pltpu.sync_copy(normp.at[pl.ds(sb, se)], normb)

    def prep(g, b):
        g0 = (g % STRIPE_G) * GRP
        for j in range(GRP // LANES):
            off = g0 + j * LANES
            et_v = etb[pl.ds(off, LANES)]
            s_v = srcb[pl.ds(off, LANES)]
            fibs[b][pl.ds(j * LANES, LANES)] = \
                (et_v * n_nodes + s_v) * 2 + cc
            dstgs[b][pl.ds(j * LANES, LANES)] = dstb[pl.ds(off, LANES)]
            normgs[b][pl.ds(j * LANES, LANES)] = normb[pl.ds(off, LANES)]

    def start_gather(b):
        pltpu.async_copy(h2.at[fibs[b]], rowss[b], semgs[b])

    def wait_gather(b):
        pltpu.make_async_copy(h2.at[fibs[b]], rowss[b], semgs[b]).wait()

    def scale(b):
        rows = rowss[b]
        normg = normgs[b]

        @plsc.parallel_loop(0, GRP // LANES, unroll=2)
        def _(q):
            nv = normg[pl.ds(q * LANES, LANES)]
            i0 = q * LANES
            for l in range(LANES):
                nbc = lax.gather(
                    nv, jnp.full((LANES, 1), l, jnp.int32),
                    lax.GatherDimensionNumbers(
                        offset_dims=(), collapsed_slice_dims=(0,),
                        start_index_map=(0,)),
                    (1,), mode=lax.GatherScatterMode.PROMISE_IN_BOUNDS)
                for j in range(half // LANES):
                    v = rows[i0 + l, pl.ds(j * LANES, LANES)]
                    rows[i0 + l, pl.ds(j * LANES, LANES)] = v * nbc

    def start_scatter(b):
        pltpu.async_copy(rowss[b], acc.at[dstgs[b]], semss[b], add=True)

    def wait_scatter(b):
        pltpu.make_async_copy(rowss[b], acc.at[dstgs[b]], semss[b]).wait()

    stage(0)
    prep(0, 0)
    start_gather(0)
    steady = gt - 1
    assert steady % 3 == 0

    def triple(i3, _):
        for k in range(3):
            i = i3 + k
            b = k % 3

            @pl.when(i >= 2)
            def _():
                wait_scatter((b + 1) % 3)

            @pl.when((i + 1) % STRIPE_G == 0)
            def _():
                stage((i + 1) // STRIPE_G)
            prep(i + 1, (b + 1) % 3)
            start_gather((b + 1) % 3)
            wait_gather(b)
            scale(b)
            start_scatter(b)
        return 0

    lax.fori_loop(0, steady // 3, lambda t, c: triple(t * 3, c), 0)
    bl = (gt - 1) % 3
    wait_scatter((bl + 1) % 3)
    wait_gather(bl)
    scale(bl)
    start_scatter(bl)
    wait_scatter((bl + 2) % 3)
    wait_scatter(bl)
    plsc.subcore_barrier()

    pltpu.sync_copy(acc.at[pl.ds(base, rows_per_tile)],
                    out.at[pl.ds(base, rows_per_tile), cc])


def kernel(x, edge_index, etypes, norm, weight, w_comp, h_bias):
    n, in_feat = x.shape
    out_feat = weight.shape[2]
    nrels = w_comp.shape[0]
    e = etypes.shape[0]
    half = out_feat // 2

    h_all = _compute_h_all(x, weight, w_comp)
    h2 = h_all.reshape(nrels * n * 2, half)

    chunk = NS * STRIPE_G * GRP
    e_pad = ((e + chunk - 1) // chunk) * chunk
    pad = e_pad - e
    pad_iota = jnp.arange(pad, dtype=jnp.int32) % n
    srcp = jnp.concatenate([edge_index[0].astype(jnp.int32), pad_iota])
    dstp = jnp.concatenate([edge_index[1].astype(jnp.int32), pad_iota])
    etp = jnp.concatenate([etypes.astype(jnp.int32),
                           jnp.zeros((pad,), jnp.int32)])
    normp = jnp.concatenate([norm.reshape(e).astype(jnp.float32),
                             jnp.zeros((pad,), jnp.float32)])
    biast = jnp.broadcast_to(h_bias.reshape(2, 1, half), (2, GRP, half))
    gt = e_pad // (NS * GRP)

    mesh = plsc.VectorSubcoreMesh(core_axis_name="c", subcore_axis_name="s")
    sc_call = functools.partial(
        pl.kernel,
        out_type=jax.ShapeDtypeStruct((n, 2, half), jnp.float32),
        mesh=mesh,
        scratch_types=(
            [pltpu.VMEM_SHARED((n, half), jnp.float32)]
            + [pltpu.VMEM((STRIPE_G * GRP,), jnp.int32)] * 3
            + [pltpu.VMEM((STRIPE_G * GRP,), jnp.float32)]
            + [pltpu.VMEM((GRP,), jnp.int32)] * 6
            + [pltpu.VMEM((GRP,), jnp.float32)] * 3
            + [pltpu.VMEM((GRP, half), jnp.float32)] * 3
            + [pltpu.SemaphoreType.DMA] * 6
        ),
        compiler_params=pltpu.CompilerParams(use_tc_tiling_on_sc=False),
    )(functools.partial(_sc_body, n, half, gt))
    out3 = sc_call(h2, srcp, etp, dstp, normp, biast)
    return out3.reshape(n, out_feat)


# speedup vs baseline: 7.6029x; 1.0479x over previous
import functools

import jax
import jax.numpy as jnp
from jax import lax
from jax.experimental import pallas as pl
from jax.experimental.pallas import tpu as pltpu
from jax.experimental.pallas import tpu_sc as plsc

NC = 2
NS = 16
LANES = 16
GRP = 128
STRIPE_G = 32


def _hall_body(wc_ref, w_ref, x_ref, out_ref):
    r = pl.program_id(1)
    nb = w_ref.shape[0]
    w_r = wc_ref[r, 0] * w_ref[0]
    for b in range(1, nb):
        w_r = w_r + wc_ref[r, b] * w_ref[b]
    out_ref[0] = jnp.dot(x_ref[...], w_r, preferred_element_type=jnp.float32)


def _compute_h_all(x, weight, w_comp):
    n, in_feat = x.shape
    nbases, _, out_feat = weight.shape
    nrels = w_comp.shape[0]
    bn = 2000
    grid = (n // bn, nrels)
    return pl.pallas_call(
        _hall_body,
        grid=grid,
        in_specs=[
            pl.BlockSpec(memory_space=pltpu.SMEM),
            pl.BlockSpec((nbases, in_feat, out_feat), lambda i, r: (0, 0, 0)),
            pl.BlockSpec((bn, in_feat), lambda i, r: (i, 0)),
        ],
        out_specs=pl.BlockSpec((1, bn, out_feat), lambda i, r: (r, i, 0)),
        out_shape=jax.ShapeDtypeStruct((nrels, n, out_feat), jnp.float32),
    )(w_comp, weight, x)


def _sc_body(n_nodes, half, gt, h2, srcp, etp, dstp, normp, biast, out,
             acc, srcb, etb, dstb, normb,
             fib0, fib1, fib2, dstg0, dstg1, dstg2,
             normg0, normg1, normg2, rows0, rows1, rows2,
             semg0, semg1, semg2, sems0, sems1, sems2):
    cc = lax.axis_index("c")
    ss = lax.axis_index("s")
    fibs = (fib0, fib1, fib2)
    dstgs = (dstg0, dstg1, dstg2)
    normgs = (normg0, normg1, normg2)
    rowss = (rows0, rows1, rows2)
    semgs = (semg0, semg1, semg2)
    semss = (sems0, sems1, sems2)

    rows_per_tile = n_nodes // NS
    base = ss * rows_per_tile
    pltpu.sync_copy(biast.at[cc], rows0)
    full, rem = divmod(rows_per_tile, GRP)
    for k in range(full):
        pltpu.sync_copy(rows0, acc.at[pl.ds(base + k * GRP, GRP)])
    if rem:
        pltpu.sync_copy(rows0.at[pl.ds(0, rem)],
                        acc.at[pl.ds(base + full * GRP, rem)])
    plsc.subcore_barrier()

    te = gt * GRP
    tile_e0 = ss * te
    se = STRIPE_G * GRP

    def stage(s):
        sb = tile_e0 + s * se
        pltpu.sync_copy(srcp.at[pl.ds(sb, se)], srcb)
        pltpu.sync_copy(etp.at[pl.ds(sb, se)], etb)
        pltpu.sync_copy(dstp.at[pl.ds(sb, se)], dstb)
        pltpu.sync_copy(normp.at[pl.ds(sb, se)], normb)

    def prep(g, b):
        g0 = (g % STRIPE_G) * GRP
        for j in range(GRP // LANES):
            off = g0 + j * LANES
            et_v = etb[pl.ds(off, LANES)]
            s_v = srcb[pl.ds(off, LANES)]
            fibs[b][pl.ds(j * LANES, LANES)] = \
                (et_v * n_nodes + s_v) * 2 + cc
            dstgs[b][pl.ds(j * LANES, LANES)] = dstb[pl.ds(off, LANES)]
            normgs[b][pl.ds(j * LANES, LANES)] = normb[pl.ds(off, LANES)]

    def start_gather(b):
        pltpu.async_copy(h2.at[fibs[b]], rowss[b], semgs[b])

    def wait_gather(b):
        pltpu.make_async_copy(h2.at[fibs[b]], rowss[b], semgs[b]).wait()

    def scale(b):
        rows = rowss[b]
        normg = normgs[b]

        @plsc.parallel_loop(0, GRP // LANES, unroll=2)
        def _(q):
            nv = normg[pl.ds(q * LANES, LANES)]
            i0 = q * LANES
            for l in range(LANES):
                nbc = lax.gather(
                    nv, jnp.full((LANES, 1), l, jnp.int32),
                    lax.GatherDimensionNumbers(
                        offset_dims=(), collapsed_slice_dims=(0,),
                        start_index_map=(0,)),
                    (1,), mode=lax.GatherScatterMode.PROMISE_IN_BOUNDS)
                for j in range(half // LANES):
                    v = rows[i0 + l, pl.ds(j * LANES, LANES)]
                    rows[i0 + l, pl.ds(j * LANES, LANES)] = v * nbc

    def start_scatter(b):
        pltpu.async_copy(rowss[b], acc.at[dstgs[b]], semss[b], add=True)

    def wait_scatter(b):
        pltpu.make_async_copy(rowss[b], acc.at[dstgs[b]], semss[b]).wait()

    stage(0)
    prep(0, 0)
    start_gather(0)
    steady = gt - 1
    assert steady % 3 == 0

    def triple(i3, _):
        for k in range(3):
            i = i3 + k
            b = k % 3

            @pl.when(i >= 2)
            def _():
                wait_scatter((b + 1) % 3)

            @pl.when((i + 1) % STRIPE_G == 0)
            def _():
                stage((i + 1) // STRIPE_G)
            prep(i + 1, (b + 1) % 3)
            start_gather((b + 1) % 3)
            wait_gather(b)
            scale(b)
            start_scatter(b)
        return 0

    lax.fori_loop(0, steady // 3, lambda t, c: triple(t * 3, c), 0)
    bl = (gt - 1) % 3
    wait_scatter((bl + 1) % 3)
    wait_gather(bl)
    scale(bl)
    start_scatter(bl)
    wait_scatter((bl + 2) % 3)
    wait_scatter(bl)
    plsc.subcore_barrier()

    pltpu.sync_copy(acc.at[pl.ds(base, rows_per_tile)],
                    out.at[pl.ds(base, rows_per_tile), cc])


def kernel(x, edge_index, etypes, norm, weight, w_comp, h_bias):
    n, in_feat = x.shape
    out_feat = weight.shape[2]
    nrels = w_comp.shape[0]
    e = etypes.shape[0]
    half = out_feat // 2

    h_all = _compute_h_all(x, weight, w_comp)
    h2 = h_all.reshape(nrels * n * 2, half)

    chunk = NS * STRIPE_G * GRP
    e_pad = ((e + chunk - 1) // chunk) * chunk
    pad = e_pad - e
    pad_iota = jnp.arange(pad, dtype=jnp.int32) % n
    srcp = jnp.concatenate([edge_index[0].astype(jnp.int32), pad_iota])
    dstp = jnp.concatenate([edge_index[1].astype(jnp.int32), pad_iota])
    etp = jnp.concatenate([etypes.astype(jnp.int32),
                           jnp.zeros((pad,), jnp.int32)])
    normp = jnp.concatenate([norm.reshape(e).astype(jnp.float32),
                             jnp.zeros((pad,), jnp.float32)])
    biast = jnp.broadcast_to(h_bias.reshape(2, 1, half), (2, GRP, half))
    gt = e_pad // (NS * GRP)

    mesh = plsc.VectorSubcoreMesh(core_axis_name="c", subcore_axis_name="s")
    sc_call = functools.partial(
        pl.kernel,
        out_type=jax.ShapeDtypeStruct((n, 2, half), jnp.float32),
        mesh=mesh,
        scratch_types=(
            [pltpu.VMEM_SHARED((n, half), jnp.float32)]
            + [pltpu.VMEM((STRIPE_G * GRP,), jnp.int32)] * 3
            + [pltpu.VMEM((STRIPE_G * GRP,), jnp.float32)]
            + [pltpu.VMEM((GRP,), jnp.int32)] * 6
            + [pltpu.VMEM((GRP,), jnp.float32)] * 3
            + [pltpu.VMEM((GRP, half), jnp.float32)] * 3
            + [pltpu.SemaphoreType.DMA] * 6
        ),
        compiler_params=pltpu.CompilerParams(use_tc_tiling_on_sc=False),
    )(functools.partial(_sc_body, n, half, gt))
    out3 = sc_call(h2, srcp, etp, dstp, normp, biast)
    return out3.reshape(n, out_feat)


# speedup vs baseline: 7.9484x; 1.0454x over previous
import functools

import jax
import jax.numpy as jnp
from jax import lax
from jax.experimental import pallas as pl
from jax.experimental.pallas import tpu as pltpu
from jax.experimental.pallas import tpu_sc as plsc

NC = 2
NS = 16
LANES = 16
GRP = 256
STRIPE_G = 16


def _hall_body(wc_ref, w_ref, x_ref, out_ref):
    r = pl.program_id(1)
    nb = w_ref.shape[0]
    w_r = wc_ref[r, 0] * w_ref[0]
    for b in range(1, nb):
        w_r = w_r + wc_ref[r, b] * w_ref[b]
    out_ref[0] = jnp.dot(x_ref[...], w_r, preferred_element_type=jnp.float32)


def _compute_h_all(x, weight, w_comp):
    n, in_feat = x.shape
    nbases, _, out_feat = weight.shape
    nrels = w_comp.shape[0]
    bn = 2000
    grid = (n // bn, nrels)
    return pl.pallas_call(
        _hall_body,
        grid=grid,
        in_specs=[
            pl.BlockSpec(memory_space=pltpu.SMEM),
            pl.BlockSpec((nbases, in_feat, out_feat), lambda i, r: (0, 0, 0)),
            pl.BlockSpec((bn, in_feat), lambda i, r: (i, 0)),
        ],
        out_specs=pl.BlockSpec((1, bn, out_feat), lambda i, r: (r, i, 0)),
        out_shape=jax.ShapeDtypeStruct((nrels, n, out_feat), jnp.float32),
    )(w_comp, weight, x)


def _sc_body(n_nodes, half, gt, h2, srcp, etp, dstp, normp, biast, out,
             acc, srcb, etb, dstb, normb,
             fib0, fib1, fib2, dstg0, dstg1, dstg2,
             normg0, normg1, normg2, rows0, rows1, rows2,
             semg0, semg1, semg2, sems0, sems1, sems2):
    cc = lax.axis_index("c")
    ss = lax.axis_index("s")
    fibs = (fib0, fib1, fib2)
    dstgs = (dstg0, dstg1, dstg2)
    normgs = (normg0, normg1, normg2)
    rowss = (rows0, rows1, rows2)
    semgs = (semg0, semg1, semg2)
    semss = (sems0, sems1, sems2)

    rows_per_tile = n_nodes // NS
    base = ss * rows_per_tile
    pltpu.sync_copy(biast.at[cc], rows0)
    full, rem = divmod(rows_per_tile, GRP)
    for k in range(full):
        pltpu.sync_copy(rows0, acc.at[pl.ds(base + k * GRP, GRP)])
    if rem:
        pltpu.sync_copy(rows0.at[pl.ds(0, rem)],
                        acc.at[pl.ds(base + full * GRP, rem)])
    plsc.subcore_barrier()

    te = gt * GRP
    tile_e0 = ss * te
    se = STRIPE_G * GRP

    def stage(s):
        sb = tile_e0 + s * se
        pltpu.sync_copy(srcp.at[pl.ds(sb, se)], srcb)
        pltpu.sync_copy(etp.at[pl.ds(sb, se)], etb)
        pltpu.sync_copy(dstp.at[pl.ds(sb, se)], dstb)
        pltpu.sync_copy(normp.at[pl.ds(sb, se)], normb)

    def prep(g, b):
        g0 = (g % STRIPE_G) * GRP
        for j in range(GRP // LANES):
            off = g0 + j * LANES
            et_v = etb[pl.ds(off, LANES)]
            s_v = srcb[pl.ds(off, LANES)]
            fibs[b][pl.ds(j * LANES, LANES)] = \
                (et_v * n_nodes + s_v) * 2 + cc
            dstgs[b][pl.ds(j * LANES, LANES)] = dstb[pl.ds(off, LANES)]
            normgs[b][pl.ds(j * LANES, LANES)] = normb[pl.ds(off, LANES)]

    def start_gather(b):
        pltpu.async_copy(h2.at[fibs[b]], rowss[b], semgs[b])

    def wait_gather(b):
        pltpu.make_async_copy(h2.at[fibs[b]], rowss[b], semgs[b]).wait()

    def scale(b):
        rows = rowss[b]
        normg = normgs[b]

        @plsc.parallel_loop(0, GRP // LANES, unroll=2)
        def _(q):
            nv = normg[pl.ds(q * LANES, LANES)]
            i0 = q * LANES
            for l in range(LANES):
                nbc = lax.gather(
                    nv, jnp.full((LANES, 1), l, jnp.int32),
                    lax.GatherDimensionNumbers(
                        offset_dims=(), collapsed_slice_dims=(0,),
                        start_index_map=(0,)),
                    (1,), mode=lax.GatherScatterMode.PROMISE_IN_BOUNDS)
                for j in range(half // LANES):
                    v = rows[i0 + l, pl.ds(j * LANES, LANES)]
                    rows[i0 + l, pl.ds(j * LANES, LANES)] = v * nbc

    def start_scatter(b):
        pltpu.async_copy(rowss[b], acc.at[dstgs[b]], semss[b], add=True)

    def wait_scatter(b):
        pltpu.make_async_copy(rowss[b], acc.at[dstgs[b]], semss[b]).wait()

    stage(0)
    prep(0, 0)
    start_gather(0)
    steady = gt - 1

    def body(i, b, static):
        if static:
            if i >= 2:
                wait_scatter((b + 1) % 3)
            if (i + 1) % STRIPE_G == 0:
                stage((i + 1) // STRIPE_G)
        else:
            @pl.when(i >= 2)
            def _():
                wait_scatter((b + 1) % 3)

            @pl.when((i + 1) % STRIPE_G == 0)
            def _():
                stage((i + 1) // STRIPE_G)
        prep(i + 1, (b + 1) % 3)
        start_gather((b + 1) % 3)
        wait_gather(b)
        scale(b)
        start_scatter(b)

    def triple(i3, _):
        for k in range(3):
            body(i3 + k, k, False)
        return 0

    lax.fori_loop(0, steady // 3, lambda t, c: triple(t * 3, c), 0)
    for i in range(steady - steady % 3, steady):
        body(i, i % 3, True)
    bl = (gt - 1) % 3
    wait_scatter((bl + 1) % 3)
    wait_gather(bl)
    scale(bl)
    start_scatter(bl)
    wait_scatter((bl + 2) % 3)
    wait_scatter(bl)
    plsc.subcore_barrier()

    pltpu.sync_copy(acc.at[pl.ds(base, rows_per_tile)],
                    out.at[pl.ds(base, rows_per_tile), cc])


def kernel(x, edge_index, etypes, norm, weight, w_comp, h_bias):
    n, in_feat = x.shape
    out_feat = weight.shape[2]
    nrels = w_comp.shape[0]
    e = etypes.shape[0]
    half = out_feat // 2

    h_all = _compute_h_all(x, weight, w_comp)
    h2 = h_all.reshape(nrels * n * 2, half)

    chunk = NS * STRIPE_G * GRP
    e_pad = ((e + chunk - 1) // chunk) * chunk
    pad = e_pad - e
    pad_iota = jnp.arange(pad, dtype=jnp.int32) % n
    srcp = jnp.concatenate([edge_index[0].astype(jnp.int32), pad_iota])
    dstp = jnp.concatenate([edge_index[1].astype(jnp.int32), pad_iota])
    etp = jnp.concatenate([etypes.astype(jnp.int32),
                           jnp.zeros((pad,), jnp.int32)])
    normp = jnp.concatenate([norm.reshape(e).astype(jnp.float32),
                             jnp.zeros((pad,), jnp.float32)])
    biast = jnp.broadcast_to(h_bias.reshape(2, 1, half), (2, GRP, half))
    gt = e_pad // (NS * GRP)

    mesh = plsc.VectorSubcoreMesh(core_axis_name="c", subcore_axis_name="s")
    sc_call = functools.partial(
        pl.kernel,
        out_type=jax.ShapeDtypeStruct((n, 2, half), jnp.float32),
        mesh=mesh,
        scratch_types=(
            [pltpu.VMEM_SHARED((n, half), jnp.float32)]
            + [pltpu.VMEM((STRIPE_G * GRP,), jnp.int32)] * 3
            + [pltpu.VMEM((STRIPE_G * GRP,), jnp.float32)]
            + [pltpu.VMEM((GRP,), jnp.int32)] * 6
            + [pltpu.VMEM((GRP,), jnp.float32)] * 3
            + [pltpu.VMEM((GRP, half), jnp.float32)] * 3
            + [pltpu.SemaphoreType.DMA] * 6
        ),
        compiler_params=pltpu.CompilerParams(use_tc_tiling_on_sc=False),
    )(functools.partial(_sc_body, n, half, gt))
    out3 = sc_call(h2, srcp, etp, dstp, normp, biast)
    return out3.reshape(n, out_feat)


# speedup vs baseline: 9.4337x; 1.1869x over previous
import functools

import jax
import jax.numpy as jnp
from jax import lax
from jax.experimental import pallas as pl
from jax.experimental.pallas import tpu as pltpu
from jax.experimental.pallas import tpu_sc as plsc

NC = 2
NS = 16
LANES = 16
GRP = 256
STRIPE_G = 16


def _hall_body(wc_ref, w_ref, x_ref, out_ref):
    r = pl.program_id(1)
    nb = w_ref.shape[0]
    w_r = wc_ref[r, 0] * w_ref[0]
    for b in range(1, nb):
        w_r = w_r + wc_ref[r, b] * w_ref[b]
    out_ref[0] = jnp.dot(x_ref[...], w_r, preferred_element_type=jnp.float32)


def _compute_h_all(x, weight, w_comp):
    n, in_feat = x.shape
    nbases, _, out_feat = weight.shape
    nrels = w_comp.shape[0]
    bn = 2000
    grid = (n // bn, nrels)
    return pl.pallas_call(
        _hall_body,
        grid=grid,
        in_specs=[
            pl.BlockSpec(memory_space=pltpu.SMEM),
            pl.BlockSpec((nbases, in_feat, out_feat), lambda i, r: (0, 0, 0)),
            pl.BlockSpec((bn, in_feat), lambda i, r: (i, 0)),
        ],
        out_specs=pl.BlockSpec((1, bn, out_feat), lambda i, r: (r, i, 0)),
        out_shape=jax.ShapeDtypeStruct((nrels, n, out_feat), jnp.float32),
    )(w_comp, weight, x)


def _sc_body(n_nodes, half, gt, h2, srcp, etp, dstp, normp, biast, out,
             acc, srcb, etb, dstb, normb,
             fib0, fib1, fib2, dstg0, dstg1, dstg2,
             normg0, normg1, normg2, rows0, rows1, rows2,
             semg0, semg1, semg2, sems0, sems1, sems2):
    cc = lax.axis_index("c")
    ss = lax.axis_index("s")
    fibs = (fib0, fib1, fib2)
    dstgs = (dstg0, dstg1, dstg2)
    normgs = (normg0, normg1, normg2)
    rowss = (rows0, rows1, rows2)
    semgs = (semg0, semg1, semg2)
    semss = (sems0, sems1, sems2)

    rows_per_tile = n_nodes // NS
    base = ss * rows_per_tile
    pltpu.sync_copy(biast.at[cc], rows0)
    full, rem = divmod(rows_per_tile, GRP)
    for k in range(full):
        pltpu.sync_copy(rows0, acc.at[pl.ds(base + k * GRP, GRP)])
    if rem:
        pltpu.sync_copy(rows0.at[pl.ds(0, rem)],
                        acc.at[pl.ds(base + full * GRP, rem)])
    plsc.subcore_barrier()

    te = gt * GRP
    tile_e0 = ss * te
    se = STRIPE_G * GRP

    def stage(s):
        sb = tile_e0 + s * se
        pltpu.sync_copy(srcp.at[pl.ds(sb, se)], srcb)
        pltpu.sync_copy(etp.at[pl.ds(sb, se)], etb)
        pltpu.sync_copy(dstp.at[pl.ds(sb, se)], dstb)
        pltpu.sync_copy(normp.at[pl.ds(sb, se)], normb)

    def prep(g, b):
        g0 = (g % STRIPE_G) * GRP
        for j in range(GRP // LANES):
            off = g0 + j * LANES
            et_v = etb[pl.ds(off, LANES)]
            s_v = srcb[pl.ds(off, LANES)]
            fibs[b][pl.ds(j * LANES, LANES)] = \
                (et_v * n_nodes + s_v) * 2 + cc
            dstgs[b][pl.ds(j * LANES, LANES)] = dstb[pl.ds(off, LANES)]
            normgs[b][pl.ds(j * LANES, LANES)] = normb[pl.ds(off, LANES)]

    def start_gather(b):
        pltpu.async_copy(h2.at[fibs[b]], rowss[b], semgs[b])

    def wait_gather(b):
        pltpu.make_async_copy(h2.at[fibs[b]], rowss[b], semgs[b]).wait()

    def scale(b):
        rows = rowss[b]
        normg = normgs[b]

        @plsc.parallel_loop(0, GRP // LANES, unroll=2)
        def _(q):
            nv = normg[pl.ds(q * LANES, LANES)]
            i0 = q * LANES
            for l in range(LANES):
                nbc = lax.gather(
                    nv, jnp.full((LANES, 1), l, jnp.int32),
                    lax.GatherDimensionNumbers(
                        offset_dims=(), collapsed_slice_dims=(0,),
                        start_index_map=(0,)),
                    (1,), mode=lax.GatherScatterMode.PROMISE_IN_BOUNDS)
                for j in range(half // LANES):
                    v = rows[i0 + l, pl.ds(j * LANES, LANES)]
                    rows[i0 + l, pl.ds(j * LANES, LANES)] = v * nbc

    def start_scatter(b):
        pltpu.async_copy(rowss[b], acc.at[dstgs[b]], semss[b], add=True)

    def wait_scatter(b):
        pltpu.make_async_copy(rowss[b], acc.at[dstgs[b]], semss[b]).wait()

    stage(0)
    prep(0, 0)
    start_gather(0)
    steady = gt - 1

    def body(i, b, static):
        if static:
            if i >= 2:
                wait_scatter((b + 1) % 3)
            if (i + 1) % STRIPE_G == 0:
                stage((i + 1) // STRIPE_G)
        else:
            @pl.when(i >= 2)
            def _():
                wait_scatter((b + 1) % 3)

            @pl.when((i + 1) % STRIPE_G == 0)
            def _():
                stage((i + 1) // STRIPE_G)
        prep(i + 1, (b + 1) % 3)
        start_gather((b + 1) % 3)
        wait_gather(b)
        scale(b)
        start_scatter(b)

    def triple(i3, _):
        for k in range(3):
            body(i3 + k, k, False)
        return 0

    lax.fori_loop(0, steady // 3, lambda t, c: triple(t * 3, c), 0)
    for i in range(steady - steady % 3, steady):
        body(i, i % 3, True)
    bl = (gt - 1) % 3
    wait_scatter((bl + 1) % 3)
    wait_gather(bl)
    scale(bl)
    start_scatter(bl)
    wait_scatter((bl + 2) % 3)
    wait_scatter(bl)
    plsc.subcore_barrier()

    pltpu.sync_copy(acc.at[pl.ds(base, rows_per_tile)],
                    out.at[pl.ds(base, rows_per_tile),
                           pl.ds(cc * half, half)])


def kernel(x, edge_index, etypes, norm, weight, w_comp, h_bias):
    n, in_feat = x.shape
    out_feat = weight.shape[2]
    nrels = w_comp.shape[0]
    e = etypes.shape[0]
    half = out_feat // 2

    h_all = _compute_h_all(x, weight, w_comp)
    h2 = h_all.reshape(nrels * n * 2, half)

    chunk = NS * STRIPE_G * GRP
    e_pad = ((e + chunk - 1) // chunk) * chunk
    pad = e_pad - e
    pad_iota = jnp.arange(pad, dtype=jnp.int32) % n
    srcp = jnp.concatenate([edge_index[0].astype(jnp.int32), pad_iota])
    dstp = jnp.concatenate([edge_index[1].astype(jnp.int32), pad_iota])
    etp = jnp.concatenate([etypes.astype(jnp.int32),
                           jnp.zeros((pad,), jnp.int32)])
    normp = jnp.concatenate([norm.reshape(e).astype(jnp.float32),
                             jnp.zeros((pad,), jnp.float32)])
    biast = jnp.broadcast_to(h_bias.reshape(2, 1, half), (2, GRP, half))
    gt = e_pad // (NS * GRP)

    mesh = plsc.VectorSubcoreMesh(core_axis_name="c", subcore_axis_name="s")
    sc_call = functools.partial(
        pl.kernel,
        out_type=jax.ShapeDtypeStruct((n, out_feat), jnp.float32),
        mesh=mesh,
        scratch_types=(
            [pltpu.VMEM_SHARED((n, half), jnp.float32)]
            + [pltpu.VMEM((STRIPE_G * GRP,), jnp.int32)] * 3
            + [pltpu.VMEM((STRIPE_G * GRP,), jnp.float32)]
            + [pltpu.VMEM((GRP,), jnp.int32)] * 6
            + [pltpu.VMEM((GRP,), jnp.float32)] * 3
            + [pltpu.VMEM((GRP, half), jnp.float32)] * 3
            + [pltpu.SemaphoreType.DMA] * 6
        ),
        compiler_params=pltpu.CompilerParams(use_tc_tiling_on_sc=False),
    )(functools.partial(_sc_body, n, half, gt))
    return sc_call(h2, srcp, etp, dstp, normp, biast)
